# baseline (speedup 1.0000x reference)
import functools

import jax
import jax.numpy as jnp
from jax import lax
from jax.experimental import pallas as pl
from jax.experimental.pallas import tpu as pltpu
from jax.experimental.pallas import tpu_sc as plsc

_K = 500
_OUT_SIZE_FACTOR = 4.0
_SCORE_THRESHOLD = 0.1
_GRIDB = 2048.0

_B, _C, _H, _W = 4, 2, 512, 512
_HW = _H * _W
_CHW = _C * _HW
_NSLICE = 8
_SLICE = _CHW // _NSLICE
_NBUCK = 512
_CAND_CAP = 2048
_MERGE_CAP = 4096
_OROWS = 512
_NROWS = _B * _OROWS
_SELPAD = _NROWS + 512

_PI = 3.14159265358979
_PI_2 = 1.5707963267948966

_ATAN_C = (0.9999999981419218, -0.33333292787705715, 0.19998532263347163,
           -0.14264888583256646, 0.10958341227667072, -0.08427560725997432,
           0.05845650556360228, -0.0317490822377156, 0.011256772475624163,
           -0.001877352082647006)


def _suppress_body(x_ref, o_ref):
    x = x_ref[0]
    s = 1.0 / (1.0 + jnp.exp(-x))
    ninf = jnp.full((1, s.shape[1]), -jnp.inf, s.dtype)
    up = jnp.concatenate([s[1:], ninf], axis=0)
    dn = jnp.concatenate([ninf, s[:-1]], axis=0)
    r = jnp.maximum(jnp.maximum(s, up), dn)
    ninfc = jnp.full((r.shape[0], 1), -jnp.inf, r.dtype)
    lt = jnp.concatenate([r[:, 1:], ninfc], axis=1)
    rt = jnp.concatenate([ninfc, r[:, :-1]], axis=1)
    m = jnp.maximum(jnp.maximum(r, lt), rt)
    o_ref[0] = jnp.where(s == m, s, 0.0)


def _suppress(heatmap):
    B, C, H, W = heatmap.shape
    hm = heatmap.reshape(B * C, H, W)
    out = pl.pallas_call(
        _suppress_body,
        grid=(B * C,),
        in_specs=[pl.BlockSpec((1, H, W), lambda i: (i, 0, 0))],
        out_specs=pl.BlockSpec((1, H, W), lambda i: (i, 0, 0)),
        out_shape=jax.ShapeDtypeStruct((B * C, H, W), jnp.float32),
    )(hm)
    return out.reshape(B * C * H * W)


def _bucket_of(v):
    t = 2.0 - jnp.maximum(v, 1e-6)
    u = lax.bitcast_convert_type(t, jnp.int32)
    return lax.shift_right_logical(u, 14) & (_NBUCK - 1)


def _atan2(y, x):
    ay = jnp.abs(y)
    ax = jnp.abs(x)
    hi = jnp.maximum(ay, ax)
    lo = jnp.minimum(ay, ax)
    t = lo / jnp.maximum(hi, 1e-30)
    s2 = t * t
    p = jnp.full(t.shape, _ATAN_C[-1], jnp.float32)
    for c in reversed(_ATAN_C[:-1]):
        p = p * s2 + c
    a = t * p
    a = jnp.where(ay > ax, _PI_2 - a, a)
    a = jnp.where(x < 0.0, _PI - a, a)
    a = jnp.where(y < 0.0, -a, a)
    return a


def _select_body(sup, sc_out, ix_out,
                 data, hist, cmax, cand_v, cand_i, bhist, totals, mval, midx,
                 cnts8, cntbuf, rankb, zf, zi,
                 sh_hist, sh_cnt, sh_mval, sh_midx, sem):
    cax = lax.axis_index("c")
    sax = lax.axis_index("s")
    wid = cax * 16 + sax
    b = 2 * cax + sax // _NSLICE
    b2 = sax // _NSLICE
    j = sax % _NSLICE
    lane = lax.iota(jnp.int32, 16)
    z16i = jnp.zeros((16,), jnp.int32)
    z16f = jnp.zeros((16,), jnp.float32)

    base = b * _CHW + j * _SLICE
    dcp = pltpu.make_async_copy(sup.at[pl.ds(base, _SLICE)], data, sem)
    dcp.start()

    def _zf(i, _):
        zf[pl.ds(i * 16, 16)] = z16f
        zi[pl.ds(i * 16, 16)] = z16i
        return 0
    lax.fori_loop(0, 32, _zf, 0)

    def _zh(i, _):
        for k in range(8):
            hist[pl.ds((i * 8 + k) * 16, 16)] = z16i
        return 0
    lax.fori_loop(0, _NBUCK * 16 // 128, _zh, 0)

    pltpu.sync_copy(zf, sh_mval.at[b2, pl.ds(j * 512, 512)])
    pltpu.sync_copy(zi, sh_midx.at[b2, pl.ds(j * 512, 512)])
    pltpu.sync_copy(zf.at[pl.ds(0, 64)], sc_out.at[pl.ds(wid * 64, 64)])
    pltpu.sync_copy(zi.at[pl.ds(0, 64)], ix_out.at[pl.ds(wid * 64, 64)])

    dcp.wait()

    ones16 = jnp.ones((16,), jnp.int32)
    lane_nb = lane * _NBUCK

    def _ha(i, _):
        vs = []
        for k in range(4):
            v = data[pl.ds((i * 4 + k) * 16, 16)]
            vs.append(v)
            bkt = _bucket_of(v)
            plsc.addupdate_scatter(hist, [lane_nb + bkt], ones16,
                                   mask=v > 0.0)
        cm = jnp.max(jnp.maximum(jnp.maximum(vs[0], vs[1]),
                                 jnp.maximum(vs[2], vs[3])))
        plsc.store_scatter(cmax, [jnp.full((16,), i, jnp.int32)],
                           jnp.full((16,), cm, jnp.float32),
                           mask=lane == 0)
        return 0
    lax.fori_loop(0, _SLICE // 64, _ha, 0)

    def _hb(i, acc_):
        def _hl(l, a):
            return a + hist[pl.ds(l * _NBUCK + i * 16, 16)]
        acc = lax.fori_loop(1, 16, _hl, hist[pl.ds(i * 16, 16)])
        totals[pl.ds(i * 16, 16)] = acc
        return 0
    lax.fori_loop(0, _NBUCK // 16, _hb, 0)
    pltpu.sync_copy(totals, sh_hist.at[b2, j])
    plsc.subcore_barrier()

    pltpu.sync_copy(sh_hist.at[b2], bhist)

    def _hc(i, _):
        def _hj(l, a):
            return a + bhist[l, pl.ds(i * 16, 16)]
        acc = lax.fori_loop(1, 8, _hj, bhist[0, pl.ds(i * 16, 16)])
        totals[pl.ds(i * 16, 16)] = acc
        return 0
    lax.fori_loop(0, _NBUCK // 16, _hc, 0)

    def _sb(i, carry):
        beta_, csum_ = carry
        cs = plsc.cumsum(totals[pl.ds(i * 16, 16)]) + csum_
        hit = cs >= _OROWS
        anyh = jnp.any(hit)
        f = jnp.max(plsc.all_reduce_ffs(hit))
        cand_b = i * 16 + f
        beta_ = jnp.where((beta_ >= _NBUCK) & anyh, cand_b, beta_)
        return beta_, jnp.max(cs)
    beta, _tot = lax.fori_loop(0, _NBUCK // 16, _sb,
                               (jnp.int32(_NBUCK + 1), jnp.int32(0)))

    def _blk(blk, c0):
        c2 = c0
        for q in range(4):
            i = blk * 4 + q
            v = data[pl.ds(i * 16, 16)]
            bkt = _bucket_of(v)
            mm = (v > 0.0) & (bkt <= beta)
            nm = jnp.sum(mm.astype(jnp.int32))
            ok = c2 < _CAND_CAP - 32

            @pl.when((nm > 0) & ok)
            def _():
                pos = jnp.full((16,), c2, jnp.int32) + \
                    plsc.cumsum(mm.astype(jnp.int32)) - 1
                gidx = jnp.full((16,), j * _SLICE + i * 16,
                                jnp.int32) + lane
                plsc.store_scatter(cand_v, [pos], v, mask=mm)
                plsc.store_scatter(cand_i, [pos], gidx, mask=mm)
            c2 = c2 + jnp.where(ok, nm, 0)
        return c2

    def _cd(g, cnt):
        cmv = cmax[pl.ds(g * 16, 16)]
        qm = (cmv > 0.0) & (_bucket_of(cmv) <= beta)

        def _qc(st):
            m_, _c = st
            return jnp.any(m_)

        def _qb(st):
            m_, c_ = st
            k = jnp.max(plsc.all_reduce_ffs(m_))
            c_ = _blk(g * 16 + k, c_)
            return m_ & (lane != k), c_
        _, cnt = lax.while_loop(_qc, _qb, (qm, cnt))
        return cnt
    cnt = lax.fori_loop(0, _SLICE // 64 // 16, _cd, jnp.int32(0))

    cnt16 = (cnt + 15) // 16
    padm = lane < (cnt16 * 16 - cnt)
    pos = jnp.full((16,), cnt, jnp.int32) + lane
    plsc.store_scatter(cand_v, [pos], z16f, mask=padm)
    plsc.store_scatter(cand_i, [pos], z16i, mask=padm)

    cntbuf[pl.ds(0, 16)] = jnp.full((16,), cnt16, jnp.int32)
    pltpu.sync_copy(cntbuf, sh_cnt.at[b2, j])
    plsc.subcore_barrier()

    pltpu.sync_copy(sh_cnt.at[b2], cnts8)

    def _eb(jj, carry):
        base_, tot_ = carry
        cjj = cnts8[jj, pl.ds(0, 16)][0]
        return base_ + jnp.where(jj < j, cjj, 0), tot_ + cjj
    base16, total16 = lax.fori_loop(0, _NSLICE, _eb,
                                    (jnp.int32(0), jnp.int32(0)))

    def _ec(t, _):
        off = (base16 + t) * 16

        @pl.when(off <= _MERGE_CAP - 16)
        def _():
            pltpu.sync_copy(cand_v.at[pl.ds(t * 16, 16)],
                            sh_mval.at[b2, pl.ds(off, 16)])
            pltpu.sync_copy(cand_i.at[pl.ds(t * 16, 16)],
                            sh_midx.at[b2, pl.ds(off, 16)])
        return 0
    lax.fori_loop(0, cnt16, _ec, 0)
    plsc.subcore_barrier()

    pltpu.sync_copy(sh_mval.at[b2], mval)
    pltpu.sync_copy(sh_midx.at[b2], midx)
    total16c = jnp.minimum(total16, _MERGE_CAP // 16)

    def _rk(i, _):
        i16 = jnp.full((16,), i, jnp.int32)
        vi = plsc.load_gather(cand_v, [i16])
        xi = plsc.load_gather(cand_i, [i16])

        def _rr(tt, acc):
            for q in range(4):
                vj = mval[pl.ds((tt * 4 + q) * 16, 16)]
                xj = midx[pl.ds((tt * 4 + q) * 16, 16)]
                w = (vj > vi) | ((vj == vi) & (xj < xi))
                acc = acc + w.astype(jnp.int32)
            return acc
        acc = lax.fori_loop(0, (total16c + 3) // 4, _rr, z16i)
        plsc.store_scatter(rankb, [i16],
                           jnp.full((16,), jnp.sum(acc), jnp.int32),
                           mask=lane == 0)
        return 0
    lax.fori_loop(0, cnt16 * 16, _rk, 0)

    def _si(t, _):
        rk = rankb[pl.ds(t * 16, 16)]
        rowi = jnp.where(rk < _OROWS, b * _OROWS + rk,
                         _NROWS + ((wid * 16 + lane) & 511))
        pltpu.make_async_copy(cand_v.at[pl.ds(t * 16, 16)],
                              sc_out.at[rowi], sem).start()
        pltpu.make_async_copy(cand_i.at[pl.ds(t * 16, 16)],
                              ix_out.at[rowi], sem).start()
        return 0
    lax.fori_loop(0, cnt16, _si, 0)

    def _sd(t, _):
        pltpu.make_async_copy(cand_v.at[pl.ds(0, 16)],
                              sc_out.at[lane], sem).wait()
        pltpu.make_async_copy(cand_i.at[pl.ds(0, 16)],
                              ix_out.at[lane], sem).wait()
        return 0
    lax.fori_loop(0, cnt16, _sd, 0)


def _gather_body(scx, ixx, regf, heif, dimf, rotf, velf, out,
                 scv, ixv, ch, rowbuf, sem):
    cax = lax.axis_index("c")
    sax = lax.axis_index("s")
    wid = cax * 16 + sax
    b = wid // _NSLICE
    lane = lax.iota(jnp.int32, 16)
    base = wid * 64
    pltpu.sync_copy(scx.at[pl.ds(base, 64)], scv)
    pltpu.sync_copy(ixx.at[pl.ds(base, 64)], ixv)

    chans = ((regf, 2, 0), (regf, 2, 1), (heif, 1, 0),
             (dimf, 3, 0), (dimf, 3, 1), (dimf, 3, 2),
             (rotf, 2, 0), (rotf, 2, 1),
             (velf, 2, 0), (velf, 2, 1))
    for t in range(4):
        gi = ixv[pl.ds(t * 16, 16)]
        pix = gi & (_HW - 1)
        for k, (ref, nch, kk) in enumerate(chans):
            a = b * (nch * _HW) + kk * _HW + pix
            pltpu.make_async_copy(
                ref.at[a], ch.at[k, pl.ds(t * 16, 16)], sem).start()
    for t in range(4):
        for k in range(10):
            pltpu.make_async_copy(
                regf.at[lane], ch.at[k, pl.ds(t * 16, 16)], sem).wait()

    for t in range(4):
        gi = ixv[pl.ds(t * 16, 16)]
        sc = scv[pl.ds(t * 16, 16)]
        pix = gi & (_HW - 1)
        clsf = lax.shift_right_logical(gi, 18).astype(jnp.float32)
        ys = lax.shift_right_logical(pix, 9).astype(jnp.float32)
        xs = (pix & (_W - 1)).astype(jnp.float32)
        r0 = ch[0, pl.ds(t * 16, 16)]
        r1 = ch[1, pl.ds(t * 16, 16)]
        hei = ch[2, pl.ds(t * 16, 16)]
        e0 = jnp.exp(ch[3, pl.ds(t * 16, 16)])
        e1 = jnp.exp(ch[4, pl.ds(t * 16, 16)])
        e2 = jnp.exp(ch[5, pl.ds(t * 16, 16)])
        ang = _atan2(ch[6, pl.ds(t * 16, 16)], ch[7, pl.ds(t * 16, 16)])
        v0 = ch[8, pl.ds(t * 16, 16)]
        v1 = ch[9, pl.ds(t * 16, 16)]
        x = (xs + r0) * _OUT_SIZE_FACTOR
        y = (ys + r1) * _OUT_SIZE_FACTOR
        m = (sc > _SCORE_THRESHOLD) & (x > 0.0) & (x < _GRIDB) \
            & (y > 0.0) & (y < _GRIDB)
        scm = jnp.where(m, sc, 0.0)
        for k, val in enumerate((x, y, hei, e0, e1, e2, ang, v0, v1,
                                 scm, clsf)):
            plsc.store_scatter(
                rowbuf,
                [t * 16 + lane, jnp.full((16,), k, jnp.int32)], val)
        for k in range(11, 16):
            plsc.store_scatter(
                rowbuf,
                [t * 16 + lane, jnp.full((16,), k, jnp.int32)], jnp.zeros(
                    (16,), jnp.float32))
    pltpu.sync_copy(rowbuf, out.at[pl.ds(base, 64)])


def _mesh():
    return plsc.VectorSubcoreMesh(core_axis_name="c", subcore_axis_name="s")


def _select_sc(sup_flat):
    fn = functools.partial(
        pl.kernel,
        mesh=_mesh(),
        compiler_params=pltpu.CompilerParams(needs_layout_passes=False,
                                             use_tc_tiling_on_sc=False),
        out_type=(jax.ShapeDtypeStruct((_SELPAD,), jnp.float32),
                  jax.ShapeDtypeStruct((_SELPAD,), jnp.int32)),
        scratch_types=[
            pltpu.VMEM((_SLICE,), jnp.float32),
            pltpu.VMEM((16 * _NBUCK,), jnp.int32),
            pltpu.VMEM((_SLICE // 64,), jnp.float32),
            pltpu.VMEM((_CAND_CAP,), jnp.float32),
            pltpu.VMEM((_CAND_CAP,), jnp.int32),
            pltpu.VMEM((_NSLICE, _NBUCK), jnp.int32),
            pltpu.VMEM((_NBUCK,), jnp.int32),
            pltpu.VMEM((_MERGE_CAP,), jnp.float32),
            pltpu.VMEM((_MERGE_CAP,), jnp.int32),
            pltpu.VMEM((_NSLICE, 16), jnp.int32),
            pltpu.VMEM((16,), jnp.int32),
            pltpu.VMEM((_CAND_CAP,), jnp.int32),
            pltpu.VMEM((512,), jnp.float32),
            pltpu.VMEM((512,), jnp.int32),
            pltpu.VMEM_SHARED((2, _NSLICE, _NBUCK), jnp.int32),
            pltpu.VMEM_SHARED((2, _NSLICE, 16), jnp.int32),
            pltpu.VMEM_SHARED((2, _MERGE_CAP), jnp.float32),
            pltpu.VMEM_SHARED((2, _MERGE_CAP), jnp.int32),
            pltpu.SemaphoreType.DMA,
        ],
    )(_select_body)
    return fn(sup_flat)


def _gather_sc(scx, ixx, regf, heif, dimf, rotf, velf):
    fn = functools.partial(
        pl.kernel,
        mesh=_mesh(),
        compiler_params=pltpu.CompilerParams(needs_layout_passes=False,
                                             use_tc_tiling_on_sc=False),
        out_type=jax.ShapeDtypeStruct((_NROWS, 16), jnp.float32),
        scratch_types=[
            pltpu.VMEM((64,), jnp.float32),
            pltpu.VMEM((64,), jnp.int32),
            pltpu.VMEM((10, 64), jnp.float32),
            pltpu.VMEM((64, 16), jnp.float32),
            pltpu.SemaphoreType.DMA,
        ],
    )(_gather_body)
    return fn(scx, ixx, regf, heif, dimf, rotf, velf)


def _flat(x):
    return jnp.minimum(x.reshape(-1), jnp.float32(3.4e38))


def kernel(heatmap, reg, height, dim, rot, vel):
    sup = _suppress(heatmap)
    scx, ixx = _select_sc(sup)
    res = _gather_sc(scx, ixx, _flat(reg), _flat(height),
                     _flat(dim), _flat(rot), _flat(vel))
    return res.reshape(_B, _OROWS, 16)[:, :_K, :11]


# speedup vs baseline: 1.2739x; 1.2739x over previous
import functools

import jax
import jax.numpy as jnp
from jax import lax
from jax.experimental import pallas as pl
from jax.experimental.pallas import tpu as pltpu
from jax.experimental.pallas import tpu_sc as plsc

_K = 500
_OUT_SIZE_FACTOR = 4.0
_SCORE_THRESHOLD = 0.1
_GRIDB = 2048.0

_B, _C, _H, _W = 4, 2, 512, 512
_HW = _H * _W
_CHW = _C * _HW
_NSLICE = 8
_SLICE = _CHW // _NSLICE
_NBUCK = 512
_CAND_CAP = 2048
_MERGE_CAP = 4096
_OROWS = 512
_NROWS = _B * _OROWS
_SELPAD = _NROWS + 512

_PI = 3.14159265358979
_PI_2 = 1.5707963267948966

_ATAN_C = (0.9999999981419218, -0.33333292787705715, 0.19998532263347163,
           -0.14264888583256646, 0.10958341227667072, -0.08427560725997432,
           0.05845650556360228, -0.0317490822377156, 0.011256772475624163,
           -0.001877352082647006)


def _suppress_body(x_ref, o_ref):
    x = x_ref[0]
    s = 1.0 / (1.0 + jnp.exp(-x))
    ninf = jnp.full((1, s.shape[1]), -jnp.inf, s.dtype)
    up = jnp.concatenate([s[1:], ninf], axis=0)
    dn = jnp.concatenate([ninf, s[:-1]], axis=0)
    r = jnp.maximum(jnp.maximum(s, up), dn)
    ninfc = jnp.full((r.shape[0], 1), -jnp.inf, r.dtype)
    lt = jnp.concatenate([r[:, 1:], ninfc], axis=1)
    rt = jnp.concatenate([ninfc, r[:, :-1]], axis=1)
    m = jnp.maximum(jnp.maximum(r, lt), rt)
    o_ref[0] = jnp.where(s == m, s, 0.0)


def _suppress(heatmap):
    B, C, H, W = heatmap.shape
    hm = heatmap.reshape(B * C, H, W)
    out = pl.pallas_call(
        _suppress_body,
        grid=(B * C,),
        in_specs=[pl.BlockSpec((1, H, W), lambda i: (i, 0, 0))],
        out_specs=pl.BlockSpec((1, H, W), lambda i: (i, 0, 0)),
        out_shape=jax.ShapeDtypeStruct((B * C, H, W), jnp.float32),
    )(hm)
    return out.reshape(B * C * H * W)


def _bucket_of(v):
    t = 2.0 - jnp.maximum(v, 1e-6)
    u = lax.bitcast_convert_type(t, jnp.int32)
    return lax.shift_right_logical(u, 14) & (_NBUCK - 1)


def _atan2(y, x):
    ay = jnp.abs(y)
    ax = jnp.abs(x)
    hi = jnp.maximum(ay, ax)
    lo = jnp.minimum(ay, ax)
    t = lo / jnp.maximum(hi, 1e-30)
    s2 = t * t
    p = jnp.full(t.shape, _ATAN_C[-1], jnp.float32)
    for c in reversed(_ATAN_C[:-1]):
        p = p * s2 + c
    a = t * p
    a = jnp.where(ay > ax, _PI_2 - a, a)
    a = jnp.where(x < 0.0, _PI - a, a)
    a = jnp.where(y < 0.0, -a, a)
    return a


def _select_body(sup, sc_out, ix_out,
                 data, hist, cmax, cand_v, cand_i, bhist, totals, mval, midx,
                 cnts8, cntbuf, rankb, zf, zi,
                 sh_hist, sh_cnt, sh_mval, sh_midx, sh_selv, sh_seli, sem):
    cax = lax.axis_index("c")
    sax = lax.axis_index("s")
    wid = cax * 16 + sax
    b = 2 * cax + sax // _NSLICE
    b2 = sax // _NSLICE
    j = sax % _NSLICE
    lane = lax.iota(jnp.int32, 16)
    z16i = jnp.zeros((16,), jnp.int32)
    z16f = jnp.zeros((16,), jnp.float32)

    base = b * _CHW + j * _SLICE
    dcp = pltpu.make_async_copy(sup.at[pl.ds(base, _SLICE)], data, sem)
    dcp.start()

    def _zf(i, _):
        zf[pl.ds(i * 16, 16)] = z16f
        zi[pl.ds(i * 16, 16)] = z16i
        return 0
    lax.fori_loop(0, 32, _zf, 0)

    def _zh(i, _):
        for k in range(8):
            hist[pl.ds((i * 8 + k) * 16, 16)] = z16i
        return 0
    lax.fori_loop(0, _NBUCK * 16 // 128, _zh, 0)

    pltpu.sync_copy(zf, sh_mval.at[b2, pl.ds(j * 512, 512)])
    pltpu.sync_copy(zi, sh_midx.at[b2, pl.ds(j * 512, 512)])
    pltpu.sync_copy(zf.at[pl.ds(0, 128)], sh_selv.at[pl.ds(sax * 128, 128)])
    pltpu.sync_copy(zi.at[pl.ds(0, 128)], sh_seli.at[pl.ds(sax * 128, 128)])

    dcp.wait()

    ones16 = jnp.ones((16,), jnp.int32)
    lane_nb = lane * _NBUCK

    def _ha(i, _):
        vs = []
        for k in range(4):
            v = data[pl.ds((i * 4 + k) * 16, 16)]
            vs.append(v)
            bkt = _bucket_of(v)
            plsc.addupdate_scatter(hist, [lane_nb + bkt], ones16,
                                   mask=v > 0.0)
        cm = jnp.max(jnp.maximum(jnp.maximum(vs[0], vs[1]),
                                 jnp.maximum(vs[2], vs[3])))
        plsc.store_scatter(cmax, [jnp.full((16,), i, jnp.int32)],
                           jnp.full((16,), cm, jnp.float32),
                           mask=lane == 0)
        return 0
    lax.fori_loop(0, _SLICE // 64, _ha, 0)

    def _hb(i, acc_):
        def _hl(l, a):
            return a + hist[pl.ds(l * _NBUCK + i * 16, 16)]
        acc = lax.fori_loop(1, 16, _hl, hist[pl.ds(i * 16, 16)])
        totals[pl.ds(i * 16, 16)] = acc
        return 0
    lax.fori_loop(0, _NBUCK // 16, _hb, 0)
    pltpu.sync_copy(totals, sh_hist.at[b2, j])
    plsc.subcore_barrier()

    pltpu.sync_copy(sh_hist.at[b2], bhist)

    def _hc(i, _):
        def _hj(l, a):
            return a + bhist[l, pl.ds(i * 16, 16)]
        acc = lax.fori_loop(1, 8, _hj, bhist[0, pl.ds(i * 16, 16)])
        totals[pl.ds(i * 16, 16)] = acc
        return 0
    lax.fori_loop(0, _NBUCK // 16, _hc, 0)

    def _sb(i, carry):
        beta_, csum_ = carry
        cs = plsc.cumsum(totals[pl.ds(i * 16, 16)]) + csum_
        hit = cs >= _OROWS
        anyh = jnp.any(hit)
        f = jnp.max(plsc.all_reduce_ffs(hit))
        cand_b = i * 16 + f
        beta_ = jnp.where((beta_ >= _NBUCK) & anyh, cand_b, beta_)
        return beta_, jnp.max(cs)
    beta, _tot = lax.fori_loop(0, _NBUCK // 16, _sb,
                               (jnp.int32(_NBUCK + 1), jnp.int32(0)))

    def _blk(blk, c0):
        c2 = c0
        for q in range(4):
            i = blk * 4 + q
            v = data[pl.ds(i * 16, 16)]
            bkt = _bucket_of(v)
            mm = (v > 0.0) & (bkt <= beta)
            nm = jnp.sum(mm.astype(jnp.int32))
            ok = c2 < _CAND_CAP - 32

            @pl.when((nm > 0) & ok)
            def _():
                pos = jnp.full((16,), c2, jnp.int32) + \
                    plsc.cumsum(mm.astype(jnp.int32)) - 1
                gidx = jnp.full((16,), j * _SLICE + i * 16,
                                jnp.int32) + lane
                plsc.store_scatter(cand_v, [pos], v, mask=mm)
                plsc.store_scatter(cand_i, [pos], gidx, mask=mm)
            c2 = c2 + jnp.where(ok, nm, 0)
        return c2

    def _cd(g, cnt):
        cmv = cmax[pl.ds(g * 16, 16)]
        qm = (cmv > 0.0) & (_bucket_of(cmv) <= beta)

        def _qc(st):
            m_, _c = st
            return jnp.any(m_)

        def _qb(st):
            m_, c_ = st
            k = jnp.max(plsc.all_reduce_ffs(m_))
            c_ = _blk(g * 16 + k, c_)
            return m_ & (lane != k), c_
        _, cnt = lax.while_loop(_qc, _qb, (qm, cnt))
        return cnt
    cnt = lax.fori_loop(0, _SLICE // 64 // 16, _cd, jnp.int32(0))

    cnt16 = (cnt + 15) // 16
    padm = lane < (cnt16 * 16 - cnt)
    pos = jnp.full((16,), cnt, jnp.int32) + lane
    plsc.store_scatter(cand_v, [pos], z16f, mask=padm)
    plsc.store_scatter(cand_i, [pos], z16i, mask=padm)

    cntbuf[pl.ds(0, 16)] = jnp.full((16,), cnt16, jnp.int32)
    pltpu.sync_copy(cntbuf, sh_cnt.at[b2, j])
    plsc.subcore_barrier()

    pltpu.sync_copy(sh_cnt.at[b2], cnts8)

    def _eb(jj, carry):
        base_, tot_ = carry
        cjj = cnts8[jj, pl.ds(0, 16)][0]
        return base_ + jnp.where(jj < j, cjj, 0), tot_ + cjj
    base16, total16 = lax.fori_loop(0, _NSLICE, _eb,
                                    (jnp.int32(0), jnp.int32(0)))

    def _ec(t, _):
        off = (base16 + t) * 16

        @pl.when(off <= _MERGE_CAP - 16)
        def _():
            pltpu.sync_copy(cand_v.at[pl.ds(t * 16, 16)],
                            sh_mval.at[b2, pl.ds(off, 16)])
            pltpu.sync_copy(cand_i.at[pl.ds(t * 16, 16)],
                            sh_midx.at[b2, pl.ds(off, 16)])
        return 0
    lax.fori_loop(0, cnt16, _ec, 0)
    plsc.subcore_barrier()

    pltpu.sync_copy(sh_mval.at[b2], mval)
    pltpu.sync_copy(sh_midx.at[b2], midx)
    total16c = jnp.minimum(total16, _MERGE_CAP // 16)

    def _rk(i, _):
        i16 = jnp.full((16,), i, jnp.int32)
        vi = plsc.load_gather(cand_v, [i16])
        xi = plsc.load_gather(cand_i, [i16])

        def _rr(tt, acc):
            for q in range(4):
                vj = mval[pl.ds((tt * 4 + q) * 16, 16)]
                xj = midx[pl.ds((tt * 4 + q) * 16, 16)]
                w = (vj > vi) | ((vj == vi) & (xj < xi))
                acc = acc + w.astype(jnp.int32)
            return acc
        acc = lax.fori_loop(0, (total16c + 3) // 4, _rr, z16i)
        plsc.store_scatter(rankb, [i16],
                           jnp.full((16,), jnp.sum(acc), jnp.int32),
                           mask=lane == 0)
        return 0
    lax.fori_loop(0, cnt16 * 16, _rk, 0)

    def _si(t, _):
        rk = rankb[pl.ds(t * 16, 16)]
        rowi = jnp.where(rk < _OROWS, b2 * 1024 + rk,
                         b2 * 1024 + _OROWS + ((sax * 16 + lane) & 511))
        pltpu.make_async_copy(cand_v.at[pl.ds(t * 16, 16)],
                              sh_selv.at[rowi], sem).start()
        pltpu.make_async_copy(cand_i.at[pl.ds(t * 16, 16)],
                              sh_seli.at[rowi], sem).start()
        return 0
    lax.fori_loop(0, cnt16, _si, 0)

    def _sd(t, _):
        pltpu.make_async_copy(cand_v.at[pl.ds(0, 16)],
                              sh_selv.at[lane], sem).wait()
        pltpu.make_async_copy(cand_i.at[pl.ds(0, 16)],
                              sh_seli.at[lane], sem).wait()
        return 0
    lax.fori_loop(0, cnt16, _sd, 0)
    plsc.subcore_barrier()

    q8 = sax % 8
    bb = sax // 8
    soff = bb * 1024 + q8 * 64
    doff = (2 * cax + bb) * _OROWS + q8 * 64
    pltpu.sync_copy(sh_selv.at[pl.ds(soff, 64)],
                    sc_out.at[pl.ds(doff, 64)])
    pltpu.sync_copy(sh_seli.at[pl.ds(soff, 64)],
                    ix_out.at[pl.ds(doff, 64)])


def _gather_body(scx, ixx, regf, heif, dimf, rotf, velf, out,
                 scv, ixv, ch, rowbuf, sem):
    cax = lax.axis_index("c")
    sax = lax.axis_index("s")
    wid = cax * 16 + sax
    b = wid // _NSLICE
    lane = lax.iota(jnp.int32, 16)
    base = wid * 64
    pltpu.sync_copy(scx.at[pl.ds(base, 64)], scv)
    pltpu.sync_copy(ixx.at[pl.ds(base, 64)], ixv)

    chans = ((regf, 2, 0), (regf, 2, 1), (heif, 1, 0),
             (dimf, 3, 0), (dimf, 3, 1), (dimf, 3, 2),
             (rotf, 2, 0), (rotf, 2, 1),
             (velf, 2, 0), (velf, 2, 1))
    for t in range(4):
        gi = ixv[pl.ds(t * 16, 16)]
        pix = gi & (_HW - 1)
        for k, (ref, nch, kk) in enumerate(chans):
            a = b * (nch * _HW) + kk * _HW + pix
            pltpu.make_async_copy(
                ref.at[a], ch.at[k, pl.ds(t * 16, 16)], sem).start()
    for t in range(4):
        for k in range(10):
            pltpu.make_async_copy(
                regf.at[lane], ch.at[k, pl.ds(t * 16, 16)], sem).wait()

    for t in range(4):
        gi = ixv[pl.ds(t * 16, 16)]
        sc = scv[pl.ds(t * 16, 16)]
        pix = gi & (_HW - 1)
        clsf = lax.shift_right_logical(gi, 18).astype(jnp.float32)
        ys = lax.shift_right_logical(pix, 9).astype(jnp.float32)
        xs = (pix & (_W - 1)).astype(jnp.float32)
        r0 = ch[0, pl.ds(t * 16, 16)]
        r1 = ch[1, pl.ds(t * 16, 16)]
        hei = ch[2, pl.ds(t * 16, 16)]
        e0 = jnp.exp(ch[3, pl.ds(t * 16, 16)])
        e1 = jnp.exp(ch[4, pl.ds(t * 16, 16)])
        e2 = jnp.exp(ch[5, pl.ds(t * 16, 16)])
        ang = _atan2(ch[6, pl.ds(t * 16, 16)], ch[7, pl.ds(t * 16, 16)])
        v0 = ch[8, pl.ds(t * 16, 16)]
        v1 = ch[9, pl.ds(t * 16, 16)]
        x = (xs + r0) * _OUT_SIZE_FACTOR
        y = (ys + r1) * _OUT_SIZE_FACTOR
        m = (sc > _SCORE_THRESHOLD) & (x > 0.0) & (x < _GRIDB) \
            & (y > 0.0) & (y < _GRIDB)
        scm = jnp.where(m, sc, 0.0)
        for k, val in enumerate((x, y, hei, e0, e1, e2, ang, v0, v1,
                                 scm, clsf)):
            plsc.store_scatter(
                rowbuf,
                [t * 16 + lane, jnp.full((16,), k, jnp.int32)], val)
        for k in range(11, 16):
            plsc.store_scatter(
                rowbuf,
                [t * 16 + lane, jnp.full((16,), k, jnp.int32)], jnp.zeros(
                    (16,), jnp.float32))
    pltpu.sync_copy(rowbuf, out.at[pl.ds(base, 64)])


def _mesh():
    return plsc.VectorSubcoreMesh(core_axis_name="c", subcore_axis_name="s")


def _select_sc(sup_flat):
    fn = functools.partial(
        pl.kernel,
        mesh=_mesh(),
        compiler_params=pltpu.CompilerParams(needs_layout_passes=False,
                                             use_tc_tiling_on_sc=False),
        out_type=(jax.ShapeDtypeStruct((_NROWS,), jnp.float32),
                  jax.ShapeDtypeStruct((_NROWS,), jnp.int32)),
        scratch_types=[
            pltpu.VMEM((_SLICE,), jnp.float32),
            pltpu.VMEM((16 * _NBUCK,), jnp.int32),
            pltpu.VMEM((_SLICE // 64,), jnp.float32),
            pltpu.VMEM((_CAND_CAP,), jnp.float32),
            pltpu.VMEM((_CAND_CAP,), jnp.int32),
            pltpu.VMEM((_NSLICE, _NBUCK), jnp.int32),
            pltpu.VMEM((_NBUCK,), jnp.int32),
            pltpu.VMEM((_MERGE_CAP,), jnp.float32),
            pltpu.VMEM((_MERGE_CAP,), jnp.int32),
            pltpu.VMEM((_NSLICE, 16), jnp.int32),
            pltpu.VMEM((16,), jnp.int32),
            pltpu.VMEM((_CAND_CAP,), jnp.int32),
            pltpu.VMEM((512,), jnp.float32),
            pltpu.VMEM((512,), jnp.int32),
            pltpu.VMEM_SHARED((2, _NSLICE, _NBUCK), jnp.int32),
            pltpu.VMEM_SHARED((2, _NSLICE, 16), jnp.int32),
            pltpu.VMEM_SHARED((2, _MERGE_CAP), jnp.float32),
            pltpu.VMEM_SHARED((2, _MERGE_CAP), jnp.int32),
            pltpu.VMEM_SHARED((2048,), jnp.float32),
            pltpu.VMEM_SHARED((2048,), jnp.int32),
            pltpu.SemaphoreType.DMA,
        ],
    )(_select_body)
    return fn(sup_flat)


def _gather_sc(scx, ixx, regf, heif, dimf, rotf, velf):
    fn = functools.partial(
        pl.kernel,
        mesh=_mesh(),
        compiler_params=pltpu.CompilerParams(needs_layout_passes=False,
                                             use_tc_tiling_on_sc=False),
        out_type=jax.ShapeDtypeStruct((_NROWS, 16), jnp.float32),
        scratch_types=[
            pltpu.VMEM((64,), jnp.float32),
            pltpu.VMEM((64,), jnp.int32),
            pltpu.VMEM((10, 64), jnp.float32),
            pltpu.VMEM((64, 16), jnp.float32),
            pltpu.SemaphoreType.DMA,
        ],
    )(_gather_body)
    return fn(scx, ixx, regf, heif, dimf, rotf, velf)


def _flat(x):
    return x.reshape(-1)


def kernel(heatmap, reg, height, dim, rot, vel):
    sup = _suppress(heatmap)
    scx, ixx = _select_sc(sup)
    res = _gather_sc(scx, ixx, _flat(reg), _flat(height),
                     _flat(dim), _flat(rot), _flat(vel))
    return res.reshape(_B, _OROWS, 16)[:, :_K, :11]


# speedup vs baseline: 1.7140x; 1.3455x over previous
import functools

import jax
import jax.numpy as jnp
from jax import lax
from jax.experimental import pallas as pl
from jax.experimental.pallas import tpu as pltpu
from jax.experimental.pallas import tpu_sc as plsc

_K = 500
_OUT_SIZE_FACTOR = 4.0
_SCORE_THRESHOLD = 0.1
_GRIDB = 2048.0

_B, _C, _H, _W = 4, 2, 512, 512
_HW = _H * _W
_CHW = _C * _HW
_NSLICE = 8
_SLICE = _CHW // _NSLICE
_NBUCK = 512
_CAND_CAP = 2048
_MERGE_CAP = 4096
_OROWS = 512
_NROWS = _B * _OROWS
_SELPAD = _NROWS + 512

_PI = 3.14159265358979
_PI_2 = 1.5707963267948966

_ATAN_C = (0.9999999981419218, -0.33333292787705715, 0.19998532263347163,
           -0.14264888583256646, 0.10958341227667072, -0.08427560725997432,
           0.05845650556360228, -0.0317490822377156, 0.011256772475624163,
           -0.001877352082647006)


def _suppress_body(x_ref, o_ref):
    x = x_ref[0]
    s = 1.0 / (1.0 + jnp.exp(-x))
    ninf = jnp.full((1, s.shape[1]), -jnp.inf, s.dtype)
    up = jnp.concatenate([s[1:], ninf], axis=0)
    dn = jnp.concatenate([ninf, s[:-1]], axis=0)
    r = jnp.maximum(jnp.maximum(s, up), dn)
    ninfc = jnp.full((r.shape[0], 1), -jnp.inf, r.dtype)
    lt = jnp.concatenate([r[:, 1:], ninfc], axis=1)
    rt = jnp.concatenate([ninfc, r[:, :-1]], axis=1)
    m = jnp.maximum(jnp.maximum(r, lt), rt)
    o_ref[0] = jnp.where(s == m, s, 0.0)


def _suppress(heatmap):
    B, C, H, W = heatmap.shape
    hm = heatmap.reshape(B * C, H, W)
    out = pl.pallas_call(
        _suppress_body,
        grid=(B * C,),
        in_specs=[pl.BlockSpec((1, H, W), lambda i: (i, 0, 0))],
        out_specs=pl.BlockSpec((1, H, W), lambda i: (i, 0, 0)),
        out_shape=jax.ShapeDtypeStruct((B * C, H, W), jnp.float32),
    )(hm)
    return out.reshape(B * C * H * W)


def _bucket_of(v):
    t = 2.0 - jnp.maximum(v, 1e-6)
    u = lax.bitcast_convert_type(t, jnp.int32)
    return lax.shift_right_logical(u, 14) & (_NBUCK - 1)


def _atan2(y, x):
    ay = jnp.abs(y)
    ax = jnp.abs(x)
    hi = jnp.maximum(ay, ax)
    lo = jnp.minimum(ay, ax)
    t = lo / jnp.maximum(hi, 1e-30)
    s2 = t * t
    p = jnp.full(t.shape, _ATAN_C[-1], jnp.float32)
    for c in reversed(_ATAN_C[:-1]):
        p = p * s2 + c
    a = t * p
    a = jnp.where(ay > ax, _PI_2 - a, a)
    a = jnp.where(x < 0.0, _PI - a, a)
    a = jnp.where(y < 0.0, -a, a)
    return a


def _select_body(sup, sc_out, ix_out,
                 data, hist, cmax, cand_v, cand_i, bhist, totals, mval, midx,
                 cnts8, cntbuf, rankb, zf, zi,
                 sh_hist, sh_cnt, sh_mval, sh_midx, sh_selv, sh_seli, sem):
    cax = lax.axis_index("c")
    sax = lax.axis_index("s")
    wid = cax * 16 + sax
    b = 2 * cax + sax // _NSLICE
    b2 = sax // _NSLICE
    j = sax % _NSLICE
    lane = lax.iota(jnp.int32, 16)
    z16i = jnp.zeros((16,), jnp.int32)
    z16f = jnp.zeros((16,), jnp.float32)

    base = b * _CHW + j * _SLICE
    dcp = pltpu.make_async_copy(sup.at[pl.ds(base, _SLICE)], data, sem)
    dcp.start()

    def _zf(i, _):
        zf[pl.ds(i * 16, 16)] = z16f
        zi[pl.ds(i * 16, 16)] = z16i
        return 0
    lax.fori_loop(0, 32, _zf, 0)

    def _zh(i, _):
        for k in range(8):
            hist[pl.ds((i * 8 + k) * 16, 16)] = z16i
        return 0
    lax.fori_loop(0, _NBUCK * 16 // 128, _zh, 0)

    pltpu.sync_copy(zf, sh_mval.at[b2, pl.ds(j * 512, 512)])
    pltpu.sync_copy(zi, sh_midx.at[b2, pl.ds(j * 512, 512)])
    pltpu.sync_copy(zf.at[pl.ds(0, 128)], sh_selv.at[pl.ds(sax * 128, 128)])
    pltpu.sync_copy(zi.at[pl.ds(0, 128)], sh_seli.at[pl.ds(sax * 128, 128)])

    dcp.wait()

    ones16 = jnp.ones((16,), jnp.int32)
    lane_nb = lane * _NBUCK

    def _ha(i, _):
        vs = []
        for k in range(4):
            v = data[pl.ds((i * 4 + k) * 16, 16)]
            vs.append(v)
            bkt = _bucket_of(v)
            plsc.addupdate_scatter(hist, [lane_nb + bkt], ones16,
                                   mask=v > 0.0)
        cm = jnp.max(jnp.maximum(jnp.maximum(vs[0], vs[1]),
                                 jnp.maximum(vs[2], vs[3])))
        plsc.store_scatter(cmax, [jnp.full((16,), i, jnp.int32)],
                           jnp.full((16,), cm, jnp.float32),
                           mask=lane == 0)
        return 0
    lax.fori_loop(0, _SLICE // 64, _ha, 0)

    def _hb(i, acc_):
        def _hl(l, a):
            return a + hist[pl.ds(l * _NBUCK + i * 16, 16)]
        acc = lax.fori_loop(1, 16, _hl, hist[pl.ds(i * 16, 16)])
        totals[pl.ds(i * 16, 16)] = acc
        return 0
    lax.fori_loop(0, _NBUCK // 16, _hb, 0)
    pltpu.sync_copy(totals, sh_hist.at[b2, j])
    plsc.subcore_barrier()

    pltpu.sync_copy(sh_hist.at[b2], bhist)

    def _hc(i, _):
        def _hj(l, a):
            return a + bhist[l, pl.ds(i * 16, 16)]
        acc = lax.fori_loop(1, 8, _hj, bhist[0, pl.ds(i * 16, 16)])
        totals[pl.ds(i * 16, 16)] = acc
        return 0
    lax.fori_loop(0, _NBUCK // 16, _hc, 0)

    def _sb(i, carry):
        beta_, csum_ = carry
        cs = plsc.cumsum(totals[pl.ds(i * 16, 16)]) + csum_
        hit = cs >= _OROWS
        anyh = jnp.any(hit)
        f = jnp.max(plsc.all_reduce_ffs(hit))
        cand_b = i * 16 + f
        beta_ = jnp.where((beta_ >= _NBUCK) & anyh, cand_b, beta_)
        return beta_, jnp.max(cs)
    beta, _tot = lax.fori_loop(0, _NBUCK // 16, _sb,
                               (jnp.int32(_NBUCK + 1), jnp.int32(0)))

    def _blk(blk, c0):
        c2 = c0
        for q in range(4):
            i = blk * 4 + q
            v = data[pl.ds(i * 16, 16)]
            bkt = _bucket_of(v)
            mm = (v > 0.0) & (bkt <= beta)
            nm = jnp.sum(mm.astype(jnp.int32))
            ok = c2 < _CAND_CAP - 32

            @pl.when((nm > 0) & ok)
            def _():
                pos = jnp.full((16,), c2, jnp.int32) + \
                    plsc.cumsum(mm.astype(jnp.int32)) - 1
                gidx = jnp.full((16,), j * _SLICE + i * 16,
                                jnp.int32) + lane
                plsc.store_scatter(cand_v, [pos], v, mask=mm)
                plsc.store_scatter(cand_i, [pos], gidx, mask=mm)
            c2 = c2 + jnp.where(ok, nm, 0)
        return c2

    def _cd(g, cnt):
        cmv = cmax[pl.ds(g * 16, 16)]
        qm = (cmv > 0.0) & (_bucket_of(cmv) <= beta)

        def _qc(st):
            m_, _c = st
            return jnp.any(m_)

        def _qb(st):
            m_, c_ = st
            k = jnp.max(plsc.all_reduce_ffs(m_))
            c_ = _blk(g * 16 + k, c_)
            return m_ & (lane != k), c_
        _, cnt = lax.while_loop(_qc, _qb, (qm, cnt))
        return cnt
    cnt = lax.fori_loop(0, _SLICE // 64 // 16, _cd, jnp.int32(0))

    cnt16 = (cnt + 15) // 16
    padm = lane < (cnt16 * 16 - cnt)
    pos = jnp.full((16,), cnt, jnp.int32) + lane
    plsc.store_scatter(cand_v, [pos], z16f, mask=padm)
    plsc.store_scatter(cand_i, [pos], z16i, mask=padm)

    cntbuf[pl.ds(0, 16)] = jnp.full((16,), cnt16, jnp.int32)
    pltpu.sync_copy(cntbuf, sh_cnt.at[b2, j])
    plsc.subcore_barrier()

    pltpu.sync_copy(sh_cnt.at[b2], cnts8)

    def _eb(jj, carry):
        base_, tot_ = carry
        cjj = cnts8[jj, pl.ds(0, 16)][0]
        return base_ + jnp.where(jj < j, cjj, 0), tot_ + cjj
    base16, total16 = lax.fori_loop(0, _NSLICE, _eb,
                                    (jnp.int32(0), jnp.int32(0)))

    def _ec(t, _):
        off = (base16 + t) * 16

        @pl.when(off <= _MERGE_CAP - 16)
        def _():
            pltpu.sync_copy(cand_v.at[pl.ds(t * 16, 16)],
                            sh_mval.at[b2, pl.ds(off, 16)])
            pltpu.sync_copy(cand_i.at[pl.ds(t * 16, 16)],
                            sh_midx.at[b2, pl.ds(off, 16)])
        return 0
    lax.fori_loop(0, cnt16, _ec, 0)
    plsc.subcore_barrier()

    pltpu.sync_copy(sh_mval.at[b2], mval)
    pltpu.sync_copy(sh_midx.at[b2], midx)
    total16c = jnp.minimum(total16, _MERGE_CAP // 16)

    def _rk(i, _):
        i16 = jnp.full((16,), i, jnp.int32)
        vi = plsc.load_gather(cand_v, [i16])
        xi = plsc.load_gather(cand_i, [i16])

        def _rr(tt, acc):
            for q in range(4):
                vj = mval[pl.ds((tt * 4 + q) * 16, 16)]
                xj = midx[pl.ds((tt * 4 + q) * 16, 16)]
                w = (vj > vi) | ((vj == vi) & (xj < xi))
                acc = acc + w.astype(jnp.int32)
            return acc
        acc = lax.fori_loop(0, (total16c + 3) // 4, _rr, z16i)
        plsc.store_scatter(rankb, [i16],
                           jnp.full((16,), jnp.sum(acc), jnp.int32),
                           mask=lane == 0)
        return 0
    lax.fori_loop(0, cnt16 * 16, _rk, 0)

    def _si(t, _):
        rk = rankb[pl.ds(t * 16, 16)]
        rowi = jnp.where(rk < _OROWS, b2 * 1024 + rk,
                         b2 * 1024 + _OROWS + ((sax * 16 + lane) & 511))
        pltpu.make_async_copy(cand_v.at[pl.ds(t * 16, 16)],
                              sh_selv.at[rowi], sem).start()
        pltpu.make_async_copy(cand_i.at[pl.ds(t * 16, 16)],
                              sh_seli.at[rowi], sem).start()
        return 0
    lax.fori_loop(0, cnt16, _si, 0)

    def _sd(t, _):
        pltpu.make_async_copy(cand_v.at[pl.ds(0, 16)],
                              sh_selv.at[lane], sem).wait()
        pltpu.make_async_copy(cand_i.at[pl.ds(0, 16)],
                              sh_seli.at[lane], sem).wait()
        return 0
    lax.fori_loop(0, cnt16, _sd, 0)
    plsc.subcore_barrier()

    q8 = sax % 8
    bb = sax // 8
    soff = bb * 1024 + q8 * 64
    doff = (2 * cax + bb) * _OROWS + q8 * 64
    pltpu.sync_copy(sh_selv.at[pl.ds(soff, 64)],
                    sc_out.at[pl.ds(doff, 64)])
    pltpu.sync_copy(sh_seli.at[pl.ds(soff, 64)],
                    ix_out.at[pl.ds(doff, 64)])


def _gather_body(scx, ixx, regf, heif, dimf, rotf, velf, out,
                 scv, ixv, ch, rowbuf, sem):
    cax = lax.axis_index("c")
    sax = lax.axis_index("s")
    wid = cax * 16 + sax
    b = wid // _NSLICE
    lane = lax.iota(jnp.int32, 16)
    base = wid * 64
    pltpu.sync_copy(scx.at[pl.ds(base, 64)], scv)
    pltpu.sync_copy(ixx.at[pl.ds(base, 64)], ixv)

    chans = ((regf, 2, 0), (regf, 2, 1), (heif, 1, 0),
             (dimf, 3, 0), (dimf, 3, 1), (dimf, 3, 2),
             (rotf, 2, 0), (rotf, 2, 1),
             (velf, 2, 0), (velf, 2, 1))
    for t in range(4):
        gi = ixv[pl.ds(t * 16, 16)]
        pix = gi & (_HW - 1)
        yy = lax.shift_right_logical(pix, 9)
        xx = pix & (_W - 1)
        off = lax.shift_right_logical(xx, 7) * (_H * 128) + yy * 128 \
            + (xx & 127)
        for k, (ref, nch, kk) in enumerate(chans):
            a = (b * nch + kk) * _HW + off
            pltpu.make_async_copy(
                ref.at[a], ch.at[k, pl.ds(t * 16, 16)], sem).start()
    for t in range(4):
        for k in range(10):
            pltpu.make_async_copy(
                regf.at[lane], ch.at[k, pl.ds(t * 16, 16)], sem).wait()

    for t in range(4):
        gi = ixv[pl.ds(t * 16, 16)]
        sc = scv[pl.ds(t * 16, 16)]
        pix = gi & (_HW - 1)
        clsf = lax.shift_right_logical(gi, 18).astype(jnp.float32)
        ys = lax.shift_right_logical(pix, 9).astype(jnp.float32)
        xs = (pix & (_W - 1)).astype(jnp.float32)
        r0 = ch[0, pl.ds(t * 16, 16)]
        r1 = ch[1, pl.ds(t * 16, 16)]
        hei = ch[2, pl.ds(t * 16, 16)]
        e0 = jnp.exp(ch[3, pl.ds(t * 16, 16)])
        e1 = jnp.exp(ch[4, pl.ds(t * 16, 16)])
        e2 = jnp.exp(ch[5, pl.ds(t * 16, 16)])
        ang = _atan2(ch[6, pl.ds(t * 16, 16)], ch[7, pl.ds(t * 16, 16)])
        v0 = ch[8, pl.ds(t * 16, 16)]
        v1 = ch[9, pl.ds(t * 16, 16)]
        x = (xs + r0) * _OUT_SIZE_FACTOR
        y = (ys + r1) * _OUT_SIZE_FACTOR
        m = (sc > _SCORE_THRESHOLD) & (x > 0.0) & (x < _GRIDB) \
            & (y > 0.0) & (y < _GRIDB)
        scm = jnp.where(m, sc, 0.0)
        for k, val in enumerate((x, y, hei, e0, e1, e2, ang, v0, v1,
                                 scm, clsf)):
            plsc.store_scatter(
                rowbuf,
                [t * 16 + lane, jnp.full((16,), k, jnp.int32)], val)
        for k in range(11, 16):
            plsc.store_scatter(
                rowbuf,
                [t * 16 + lane, jnp.full((16,), k, jnp.int32)], jnp.zeros(
                    (16,), jnp.float32))
    pltpu.sync_copy(rowbuf, out.at[pl.ds(base, 64)])


def _mesh():
    return plsc.VectorSubcoreMesh(core_axis_name="c", subcore_axis_name="s")


def _select_sc(sup_flat):
    fn = functools.partial(
        pl.kernel,
        mesh=_mesh(),
        compiler_params=pltpu.CompilerParams(needs_layout_passes=False,
                                             use_tc_tiling_on_sc=False),
        out_type=(jax.ShapeDtypeStruct((_NROWS,), jnp.float32),
                  jax.ShapeDtypeStruct((_NROWS,), jnp.int32)),
        scratch_types=[
            pltpu.VMEM((_SLICE,), jnp.float32),
            pltpu.VMEM((16 * _NBUCK,), jnp.int32),
            pltpu.VMEM((_SLICE // 64,), jnp.float32),
            pltpu.VMEM((_CAND_CAP,), jnp.float32),
            pltpu.VMEM((_CAND_CAP,), jnp.int32),
            pltpu.VMEM((_NSLICE, _NBUCK), jnp.int32),
            pltpu.VMEM((_NBUCK,), jnp.int32),
            pltpu.VMEM((_MERGE_CAP,), jnp.float32),
            pltpu.VMEM((_MERGE_CAP,), jnp.int32),
            pltpu.VMEM((_NSLICE, 16), jnp.int32),
            pltpu.VMEM((16,), jnp.int32),
            pltpu.VMEM((_CAND_CAP,), jnp.int32),
            pltpu.VMEM((512,), jnp.float32),
            pltpu.VMEM((512,), jnp.int32),
            pltpu.VMEM_SHARED((2, _NSLICE, _NBUCK), jnp.int32),
            pltpu.VMEM_SHARED((2, _NSLICE, 16), jnp.int32),
            pltpu.VMEM_SHARED((2, _MERGE_CAP), jnp.float32),
            pltpu.VMEM_SHARED((2, _MERGE_CAP), jnp.int32),
            pltpu.VMEM_SHARED((2048,), jnp.float32),
            pltpu.VMEM_SHARED((2048,), jnp.int32),
            pltpu.SemaphoreType.DMA,
        ],
    )(_select_body)
    return fn(sup_flat)


def _gather_sc(scx, ixx, regf, heif, dimf, rotf, velf):
    fn = functools.partial(
        pl.kernel,
        mesh=_mesh(),
        compiler_params=pltpu.CompilerParams(needs_layout_passes=False,
                                             use_tc_tiling_on_sc=False),
        out_type=jax.ShapeDtypeStruct((_NROWS, 16), jnp.float32),
        scratch_types=[
            pltpu.VMEM((64,), jnp.float32),
            pltpu.VMEM((64,), jnp.int32),
            pltpu.VMEM((10, 64), jnp.float32),
            pltpu.VMEM((64, 16), jnp.float32),
            pltpu.SemaphoreType.DMA,
        ],
    )(_gather_body)
    return fn(scx, ixx, regf, heif, dimf, rotf, velf)


def _relayout_body(r_i, h_i, d_i, ro_i, v_i, r_o, h_o, d_o, ro_o, v_o):
    r_o[0, :, 0] = r_i[0]
    h_o[0, :, 0] = h_i[0]
    d_o[0, :, 0] = d_i[0]
    ro_o[0, :, 0] = ro_i[0]
    v_o[0, :, 0] = v_i[0]


def _relayout(reg, height, dim, rot, vel):
    arrs = (reg, height, dim, rot, vel)

    def ispec(nch):
        return pl.BlockSpec((1, nch, 512, 128),
                            lambda b, xh: (b, 0, 0, xh))

    def ospec(nch):
        return pl.BlockSpec((1, nch, 1, 512, 128),
                            lambda b, xh: (b, 0, xh, 0, 0))
    outs = pl.pallas_call(
        _relayout_body,
        grid=(_B, 4),
        in_specs=[ispec(a.shape[1]) for a in arrs],
        out_specs=[ospec(a.shape[1]) for a in arrs],
        out_shape=[jax.ShapeDtypeStruct((_B, a.shape[1], 4, _H, 128),
                                        jnp.float32) for a in arrs],
    )(*arrs)
    return tuple(o.reshape(-1) for o in outs)


def kernel(heatmap, reg, height, dim, rot, vel):
    sup = _suppress(heatmap)
    scx, ixx = _select_sc(sup)
    regf, heif, dimf, rotf, velf = _relayout(reg, height, dim, rot, vel)
    res = _gather_sc(scx, ixx, regf, heif, dimf, rotf, velf)
    return res.reshape(_B, _OROWS, 16)[:, :_K, :11]


# speedup vs baseline: 1.9263x; 1.1238x over previous
import functools

import jax
import jax.numpy as jnp
from jax import lax
from jax.experimental import pallas as pl
from jax.experimental.pallas import tpu as pltpu
from jax.experimental.pallas import tpu_sc as plsc

_K = 500
_OUT_SIZE_FACTOR = 4.0
_SCORE_THRESHOLD = 0.1
_GRIDB = 2048.0

_B, _C, _H, _W = 4, 2, 512, 512
_HW = _H * _W
_CHW = _C * _HW
_NSLICE = 8
_SLICE = _CHW // _NSLICE
_NBUCK = 512
_CAND_CAP = 2048
_MERGE_CAP = 4096
_OROWS = 512
_NROWS = _B * _OROWS
_SELPAD = _NROWS + 512

_PI = 3.14159265358979
_PI_2 = 1.5707963267948966

_ATAN_C = (0.9999999981419218, -0.33333292787705715, 0.19998532263347163,
           -0.14264888583256646, 0.10958341227667072, -0.08427560725997432,
           0.05845650556360228, -0.0317490822377156, 0.011256772475624163,
           -0.001877352082647006)


def _suppress_body(x_ref, o_ref):
    x = x_ref[0]
    s = 1.0 / (1.0 + jnp.exp(-x))
    ninf = jnp.full((1, s.shape[1]), -jnp.inf, s.dtype)
    up = jnp.concatenate([s[1:], ninf], axis=0)
    dn = jnp.concatenate([ninf, s[:-1]], axis=0)
    r = jnp.maximum(jnp.maximum(s, up), dn)
    ninfc = jnp.full((r.shape[0], 1), -jnp.inf, r.dtype)
    lt = jnp.concatenate([r[:, 1:], ninfc], axis=1)
    rt = jnp.concatenate([ninfc, r[:, :-1]], axis=1)
    m = jnp.maximum(jnp.maximum(r, lt), rt)
    sup = jnp.where(s == m, s, 0.0)
    for k in range(4):
        o_ref[0, k] = sup[:, k * 128:(k + 1) * 128]


def _suppress(heatmap):
    B, C, H, W = heatmap.shape
    hm = heatmap.reshape(B * C, H, W)
    out = pl.pallas_call(
        _suppress_body,
        grid=(B * C,),
        in_specs=[pl.BlockSpec((1, H, W), lambda i: (i, 0, 0))],
        out_specs=pl.BlockSpec((1, 4, H, 128), lambda i: (i, 0, 0, 0)),
        out_shape=jax.ShapeDtypeStruct((B * C, 4, H, 128), jnp.float32),
    )(hm)
    return out.reshape(B * C * H * W)


def _bucket_of(v):
    t = 2.0 - jnp.maximum(v, 1e-6)
    u = lax.bitcast_convert_type(t, jnp.int32)
    return lax.shift_right_logical(u, 14) & (_NBUCK - 1)


def _atan2(y, x):
    ay = jnp.abs(y)
    ax = jnp.abs(x)
    hi = jnp.maximum(ay, ax)
    lo = jnp.minimum(ay, ax)
    t = lo / jnp.maximum(hi, 1e-30)
    s2 = t * t
    p = jnp.full(t.shape, _ATAN_C[-1], jnp.float32)
    for c in reversed(_ATAN_C[:-1]):
        p = p * s2 + c
    a = t * p
    a = jnp.where(ay > ax, _PI_2 - a, a)
    a = jnp.where(x < 0.0, _PI - a, a)
    a = jnp.where(y < 0.0, -a, a)
    return a


def _select_body(sup, sc_out, ix_out,
                 data, hist, cmax, cand_v, cand_i, bhist, totals, mval, midx,
                 cnts8, cntbuf, rankb, zf, zi,
                 sh_hist, sh_cnt, sh_mval, sh_midx, sh_selv, sh_seli, sem):
    cax = lax.axis_index("c")
    sax = lax.axis_index("s")
    wid = cax * 16 + sax
    b = 2 * cax + sax // _NSLICE
    b2 = sax // _NSLICE
    j = sax % _NSLICE
    lane = lax.iota(jnp.int32, 16)
    z16i = jnp.zeros((16,), jnp.int32)
    z16f = jnp.zeros((16,), jnp.float32)

    base = b * _CHW + j * _SLICE
    dcp = pltpu.make_async_copy(sup.at[pl.ds(base, _SLICE)], data, sem)
    dcp.start()

    def _zf(i, _):
        zf[pl.ds(i * 16, 16)] = z16f
        zi[pl.ds(i * 16, 16)] = z16i
        return 0
    lax.fori_loop(0, 32, _zf, 0)

    def _zh(i, _):
        for k in range(8):
            hist[pl.ds((i * 8 + k) * 16, 16)] = z16i
        return 0
    lax.fori_loop(0, _NBUCK * 16 // 128, _zh, 0)

    pltpu.sync_copy(zf, sh_mval.at[b2, pl.ds(j * 512, 512)])
    pltpu.sync_copy(zi, sh_midx.at[b2, pl.ds(j * 512, 512)])
    pltpu.sync_copy(zf.at[pl.ds(0, 128)], sh_selv.at[pl.ds(sax * 128, 128)])
    pltpu.sync_copy(zi.at[pl.ds(0, 128)], sh_seli.at[pl.ds(sax * 128, 128)])

    dcp.wait()

    ones16 = jnp.ones((16,), jnp.int32)
    lane_nb = lane * _NBUCK

    def _ha(i, _):
        vs = []
        for k in range(4):
            v = data[pl.ds((i * 4 + k) * 16, 16)]
            vs.append(v)
            bkt = _bucket_of(v)
            plsc.addupdate_scatter(hist, [lane_nb + bkt], ones16,
                                   mask=v > 0.0)
        cm = jnp.max(jnp.maximum(jnp.maximum(vs[0], vs[1]),
                                 jnp.maximum(vs[2], vs[3])))
        plsc.store_scatter(cmax, [jnp.full((16,), i, jnp.int32)],
                           jnp.full((16,), cm, jnp.float32),
                           mask=lane == 0)
        return 0
    lax.fori_loop(0, _SLICE // 64, _ha, 0)

    def _hb(i, acc_):
        def _hl(l, a):
            return a + hist[pl.ds(l * _NBUCK + i * 16, 16)]
        acc = lax.fori_loop(1, 16, _hl, hist[pl.ds(i * 16, 16)])
        totals[pl.ds(i * 16, 16)] = acc
        return 0
    lax.fori_loop(0, _NBUCK // 16, _hb, 0)
    pltpu.sync_copy(totals, sh_hist.at[b2, j])
    plsc.subcore_barrier()

    pltpu.sync_copy(sh_hist.at[b2], bhist)

    def _hc(i, _):
        def _hj(l, a):
            return a + bhist[l, pl.ds(i * 16, 16)]
        acc = lax.fori_loop(1, 8, _hj, bhist[0, pl.ds(i * 16, 16)])
        totals[pl.ds(i * 16, 16)] = acc
        return 0
    lax.fori_loop(0, _NBUCK // 16, _hc, 0)

    def _sb(i, carry):
        beta_, csum_ = carry
        cs = plsc.cumsum(totals[pl.ds(i * 16, 16)]) + csum_
        hit = cs >= _OROWS
        anyh = jnp.any(hit)
        f = jnp.max(plsc.all_reduce_ffs(hit))
        cand_b = i * 16 + f
        beta_ = jnp.where((beta_ >= _NBUCK) & anyh, cand_b, beta_)
        return beta_, jnp.max(cs)
    beta, _tot = lax.fori_loop(0, _NBUCK // 16, _sb,
                               (jnp.int32(_NBUCK + 1), jnp.int32(0)))

    def _blk(blk, c0):
        c2 = c0
        for q in range(4):
            i = blk * 4 + q
            v = data[pl.ds(i * 16, 16)]
            bkt = _bucket_of(v)
            mm = (v > 0.0) & (bkt <= beta)
            nm = jnp.sum(mm.astype(jnp.int32))
            ok = c2 < _CAND_CAP - 32

            @pl.when((nm > 0) & ok)
            def _():
                pos = jnp.full((16,), c2, jnp.int32) + \
                    plsc.cumsum(mm.astype(jnp.int32)) - 1
                local = jnp.full((16,), i * 16, jnp.int32) + lane
                yy = lax.shift_right_logical(local, 7)
                gidx = (j // 4) * _HW + yy * _W + (j % 4) * 128 \
                    + (local & 127)
                plsc.store_scatter(cand_v, [pos], v, mask=mm)
                plsc.store_scatter(cand_i, [pos], gidx, mask=mm)
            c2 = c2 + jnp.where(ok, nm, 0)
        return c2

    def _cd(g, cnt):
        cmv = cmax[pl.ds(g * 16, 16)]
        qm = (cmv > 0.0) & (_bucket_of(cmv) <= beta)

        def _qc(st):
            m_, _c = st
            return jnp.any(m_)

        def _qb(st):
            m_, c_ = st
            k = jnp.max(plsc.all_reduce_ffs(m_))
            c_ = _blk(g * 16 + k, c_)
            return m_ & (lane != k), c_
        _, cnt = lax.while_loop(_qc, _qb, (qm, cnt))
        return cnt
    cnt = lax.fori_loop(0, _SLICE // 64 // 16, _cd, jnp.int32(0))

    cnt16 = (cnt + 15) // 16
    padm = lane < (cnt16 * 16 - cnt)
    pos = jnp.full((16,), cnt, jnp.int32) + lane
    plsc.store_scatter(cand_v, [pos], z16f, mask=padm)
    plsc.store_scatter(cand_i, [pos], z16i, mask=padm)

    cntbuf[pl.ds(0, 16)] = jnp.full((16,), cnt16, jnp.int32)
    pltpu.sync_copy(cntbuf, sh_cnt.at[b2, j])
    plsc.subcore_barrier()

    pltpu.sync_copy(sh_cnt.at[b2], cnts8)

    def _eb(jj, carry):
        base_, tot_ = carry
        cjj = cnts8[jj, pl.ds(0, 16)][0]
        return base_ + jnp.where(jj < j, cjj, 0), tot_ + cjj
    base16, total16 = lax.fori_loop(0, _NSLICE, _eb,
                                    (jnp.int32(0), jnp.int32(0)))

    def _ec(t, _):
        off = (base16 + t) * 16

        @pl.when(off <= _MERGE_CAP - 16)
        def _():
            pltpu.sync_copy(cand_v.at[pl.ds(t * 16, 16)],
                            sh_mval.at[b2, pl.ds(off, 16)])
            pltpu.sync_copy(cand_i.at[pl.ds(t * 16, 16)],
                            sh_midx.at[b2, pl.ds(off, 16)])
        return 0
    lax.fori_loop(0, cnt16, _ec, 0)
    plsc.subcore_barrier()

    pltpu.sync_copy(sh_mval.at[b2], mval)
    pltpu.sync_copy(sh_midx.at[b2], midx)
    total16c = jnp.minimum(total16, _MERGE_CAP // 16)

    def _rk(i, _):
        i16 = jnp.full((16,), i, jnp.int32)
        vi = plsc.load_gather(cand_v, [i16])
        xi = plsc.load_gather(cand_i, [i16])

        def _rr(tt, acc):
            for q in range(4):
                vj = mval[pl.ds((tt * 4 + q) * 16, 16)]
                xj = midx[pl.ds((tt * 4 + q) * 16, 16)]
                w = (vj > vi) | ((vj == vi) & (xj < xi))
                acc = acc + w.astype(jnp.int32)
            return acc
        acc = lax.fori_loop(0, (total16c + 3) // 4, _rr, z16i)
        plsc.store_scatter(rankb, [i16],
                           jnp.full((16,), jnp.sum(acc), jnp.int32),
                           mask=lane == 0)
        return 0
    lax.fori_loop(0, cnt16 * 16, _rk, 0)

    def _si(t, _):
        rk = rankb[pl.ds(t * 16, 16)]
        rowi = jnp.where(rk < _OROWS, b2 * 1024 + rk,
                         b2 * 1024 + _OROWS + ((sax * 16 + lane) & 511))
        pltpu.make_async_copy(cand_v.at[pl.ds(t * 16, 16)],
                              sh_selv.at[rowi], sem).start()
        pltpu.make_async_copy(cand_i.at[pl.ds(t * 16, 16)],
                              sh_seli.at[rowi], sem).start()
        return 0
    lax.fori_loop(0, cnt16, _si, 0)

    def _sd(t, _):
        pltpu.make_async_copy(cand_v.at[pl.ds(0, 16)],
                              sh_selv.at[lane], sem).wait()
        pltpu.make_async_copy(cand_i.at[pl.ds(0, 16)],
                              sh_seli.at[lane], sem).wait()
        return 0
    lax.fori_loop(0, cnt16, _sd, 0)
    plsc.subcore_barrier()

    q8 = sax % 8
    bb = sax // 8
    soff = bb * 1024 + q8 * 64
    doff = (2 * cax + bb) * _OROWS + q8 * 64
    pltpu.sync_copy(sh_selv.at[pl.ds(soff, 64)],
                    sc_out.at[pl.ds(doff, 64)])
    pltpu.sync_copy(sh_seli.at[pl.ds(soff, 64)],
                    ix_out.at[pl.ds(doff, 64)])


def _gather_body(scx, ixx, regf, heif, dimf, rotf, velf, out,
                 scv, ixv, ch, rowbuf, sem):
    cax = lax.axis_index("c")
    sax = lax.axis_index("s")
    wid = cax * 16 + sax
    b = wid // _NSLICE
    lane = lax.iota(jnp.int32, 16)
    base = wid * 64
    pltpu.sync_copy(scx.at[pl.ds(base, 64)], scv)
    pltpu.sync_copy(ixx.at[pl.ds(base, 64)], ixv)

    chans = ((regf, 2, 0), (regf, 2, 1), (heif, 1, 0),
             (dimf, 3, 0), (dimf, 3, 1), (dimf, 3, 2),
             (rotf, 2, 0), (rotf, 2, 1),
             (velf, 2, 0), (velf, 2, 1))
    for t in range(4):
        gi = ixv[pl.ds(t * 16, 16)]
        pix = gi & (_HW - 1)
        yy = lax.shift_right_logical(pix, 9)
        xx = pix & (_W - 1)
        off = lax.shift_right_logical(xx, 7) * (_H * 128) + yy * 128 \
            + (xx & 127)
        for k, (ref, nch, kk) in enumerate(chans):
            a = (b * nch + kk) * _HW + off
            pltpu.make_async_copy(
                ref.at[a], ch.at[k, pl.ds(t * 16, 16)], sem).start()
    for t in range(4):
        for k in range(10):
            pltpu.make_async_copy(
                regf.at[lane], ch.at[k, pl.ds(t * 16, 16)], sem).wait()

    for t in range(4):
        gi = ixv[pl.ds(t * 16, 16)]
        sc = scv[pl.ds(t * 16, 16)]
        pix = gi & (_HW - 1)
        clsf = lax.shift_right_logical(gi, 18).astype(jnp.float32)
        ys = lax.shift_right_logical(pix, 9).astype(jnp.float32)
        xs = (pix & (_W - 1)).astype(jnp.float32)
        r0 = ch[0, pl.ds(t * 16, 16)]
        r1 = ch[1, pl.ds(t * 16, 16)]
        hei = ch[2, pl.ds(t * 16, 16)]
        e0 = jnp.exp(ch[3, pl.ds(t * 16, 16)])
        e1 = jnp.exp(ch[4, pl.ds(t * 16, 16)])
        e2 = jnp.exp(ch[5, pl.ds(t * 16, 16)])
        ang = _atan2(ch[6, pl.ds(t * 16, 16)], ch[7, pl.ds(t * 16, 16)])
        v0 = ch[8, pl.ds(t * 16, 16)]
        v1 = ch[9, pl.ds(t * 16, 16)]
        x = (xs + r0) * _OUT_SIZE_FACTOR
        y = (ys + r1) * _OUT_SIZE_FACTOR
        m = (sc > _SCORE_THRESHOLD) & (x > 0.0) & (x < _GRIDB) \
            & (y > 0.0) & (y < _GRIDB)
        scm = jnp.where(m, sc, 0.0)
        for k, val in enumerate((x, y, hei, e0, e1, e2, ang, v0, v1,
                                 scm, clsf)):
            plsc.store_scatter(
                rowbuf,
                [t * 16 + lane, jnp.full((16,), k, jnp.int32)], val)
        for k in range(11, 16):
            plsc.store_scatter(
                rowbuf,
                [t * 16 + lane, jnp.full((16,), k, jnp.int32)], jnp.zeros(
                    (16,), jnp.float32))
    pltpu.sync_copy(rowbuf, out.at[pl.ds(base, 64)])


def _mesh():
    return plsc.VectorSubcoreMesh(core_axis_name="c", subcore_axis_name="s")


def _select_sc(sup_flat):
    fn = functools.partial(
        pl.kernel,
        mesh=_mesh(),
        compiler_params=pltpu.CompilerParams(needs_layout_passes=False,
                                             use_tc_tiling_on_sc=False),
        out_type=(jax.ShapeDtypeStruct((_NROWS,), jnp.float32),
                  jax.ShapeDtypeStruct((_NROWS,), jnp.int32)),
        scratch_types=[
            pltpu.VMEM((_SLICE,), jnp.float32),
            pltpu.VMEM((16 * _NBUCK,), jnp.int32),
            pltpu.VMEM((_SLICE // 64,), jnp.float32),
            pltpu.VMEM((_CAND_CAP,), jnp.float32),
            pltpu.VMEM((_CAND_CAP,), jnp.int32),
            pltpu.VMEM((_NSLICE, _NBUCK), jnp.int32),
            pltpu.VMEM((_NBUCK,), jnp.int32),
            pltpu.VMEM((_MERGE_CAP,), jnp.float32),
            pltpu.VMEM((_MERGE_CAP,), jnp.int32),
            pltpu.VMEM((_NSLICE, 16), jnp.int32),
            pltpu.VMEM((16,), jnp.int32),
            pltpu.VMEM((_CAND_CAP,), jnp.int32),
            pltpu.VMEM((512,), jnp.float32),
            pltpu.VMEM((512,), jnp.int32),
            pltpu.VMEM_SHARED((2, _NSLICE, _NBUCK), jnp.int32),
            pltpu.VMEM_SHARED((2, _NSLICE, 16), jnp.int32),
            pltpu.VMEM_SHARED((2, _MERGE_CAP), jnp.float32),
            pltpu.VMEM_SHARED((2, _MERGE_CAP), jnp.int32),
            pltpu.VMEM_SHARED((2048,), jnp.float32),
            pltpu.VMEM_SHARED((2048,), jnp.int32),
            pltpu.SemaphoreType.DMA,
        ],
    )(_select_body)
    return fn(sup_flat)


def _gather_sc(scx, ixx, regf, heif, dimf, rotf, velf):
    fn = functools.partial(
        pl.kernel,
        mesh=_mesh(),
        compiler_params=pltpu.CompilerParams(needs_layout_passes=False,
                                             use_tc_tiling_on_sc=False),
        out_type=jax.ShapeDtypeStruct((_NROWS, 16), jnp.float32),
        scratch_types=[
            pltpu.VMEM((64,), jnp.float32),
            pltpu.VMEM((64,), jnp.int32),
            pltpu.VMEM((10, 64), jnp.float32),
            pltpu.VMEM((64, 16), jnp.float32),
            pltpu.SemaphoreType.DMA,
        ],
    )(_gather_body)
    return fn(scx, ixx, regf, heif, dimf, rotf, velf)


def _relayout_body(r_i, h_i, d_i, ro_i, v_i, r_o, h_o, d_o, ro_o, v_o):
    r_o[0, :, 0] = r_i[0]
    h_o[0, :, 0] = h_i[0]
    d_o[0, :, 0] = d_i[0]
    ro_o[0, :, 0] = ro_i[0]
    v_o[0, :, 0] = v_i[0]


def _relayout(reg, height, dim, rot, vel):
    arrs = (reg, height, dim, rot, vel)

    def ispec(nch):
        return pl.BlockSpec((1, nch, 512, 128),
                            lambda b, xh: (b, 0, 0, xh))

    def ospec(nch):
        return pl.BlockSpec((1, nch, 1, 512, 128),
                            lambda b, xh: (b, 0, xh, 0, 0))
    outs = pl.pallas_call(
        _relayout_body,
        grid=(_B, 4),
        in_specs=[ispec(a.shape[1]) for a in arrs],
        out_specs=[ospec(a.shape[1]) for a in arrs],
        out_shape=[jax.ShapeDtypeStruct((_B, a.shape[1], 4, _H, 128),
                                        jnp.float32) for a in arrs],
    )(*arrs)
    return tuple(o.reshape(-1) for o in outs)


def kernel(heatmap, reg, height, dim, rot, vel):
    sup = _suppress(heatmap)
    scx, ixx = _select_sc(sup)
    regf, heif, dimf, rotf, velf = _relayout(reg, height, dim, rot, vel)
    res = _gather_sc(scx, ixx, regf, heif, dimf, rotf, velf)
    return res.reshape(_B, _OROWS, 16)[:, :_K, :11]
